# baseline (speedup 1.0000x reference)
import functools

import jax
import jax.numpy as jnp
from jax import lax
from jax.experimental import pallas as pl
from jax.experimental.pallas import tpu as pltpu
from jax.experimental.pallas import tpu_sc as plsc

NC = 2
NS = 16
LANES = 16
NW = NC * NS
CH = 128
PIPE = 2


def _make_sc_agg(np_, d, nch, want_deg):
    rt = np_ // NS
    assert rt % CH == 0

    mesh = plsc.VectorSubcoreMesh(core_axis_name="c", subcore_axis_name="s",
                                  num_cores=NC, num_subcores=NS)
    out_type = [jax.ShapeDtypeStruct((np_, d), jnp.float32)
                for _ in range(NC)]
    if want_deg:
        out_type.append(jax.ShapeDtypeStruct((NW, np_), jnp.float32))
    assert nch % PIPE == 0
    scratch = [
        pltpu.VMEM((PIPE, CH), jnp.int32),
        pltpu.VMEM((PIPE, CH), jnp.int32),
        pltpu.VMEM((PIPE, CH, d), jnp.float32),
        pltpu.VMEM_SHARED((np_, d), jnp.float32),
    ] + [pltpu.SemaphoreType.DMA] * PIPE
    if want_deg:
        scratch.append(pltpu.VMEM((np_,), jnp.float32))

    def body(h_hbm, src_hbm, dst_hbm, zrows_hbm, *refs):
        refs = list(refs)
        if want_deg:
            zdeg_hbm = refs[0]
            agg_outs = refs[1:1 + NC]
            deg_out = refs[1 + NC]
            rest = refs[2 + NC:]
            degloc = rest[-1]
            rest = rest[:-1]
        else:
            agg_outs = refs[0:NC]
            rest = refs[NC:]
        src_v, dst_v, rows_v, acc = rest[0], rest[1], rest[2], rest[3]
        sems = rest[4:4 + PIPE]

        c = lax.axis_index("c")
        s = lax.axis_index("s")
        wid = s * NC + c

        pltpu.sync_copy(zrows_hbm, rows_v.at[0])
        for kk in range(rt // CH):
            base = s * rt + kk * CH
            pltpu.sync_copy(rows_v.at[0], acc.at[pl.ds(base, CH)])
        if want_deg:
            pltpu.sync_copy(zdeg_hbm, degloc)
        plsc.subcore_barrier()

        wrow = wid * nch
        ones_vec = jnp.ones((LANES,), jnp.float32)

        @pl.loop(0, nch // PIPE)
        def edge_group(i):
            r0 = wrow + i * PIPE
            pltpu.sync_copy(src_hbm.at[pl.ds(r0, PIPE)], src_v)
            pltpu.sync_copy(dst_hbm.at[pl.ds(r0, PIPE)], dst_v)
            copies = [
                pltpu.async_copy(h_hbm.at[src_v.at[b]], rows_v.at[b],
                                 sems[b])
                for b in range(PIPE)
            ]
            for b in range(PIPE):
                copies[b].wait()
                pltpu.sync_copy(rows_v.at[b], acc.at[dst_v.at[b]], add=True)
                if want_deg:
                    for kk in range(CH // LANES):
                        idx = dst_v[b, pl.ds(kk * LANES, LANES)]
                        plsc.addupdate_scatter(degloc, [idx], ones_vec)

        plsc.subcore_barrier()

        if want_deg:
            pltpu.sync_copy(degloc, deg_out.at[wid])

        for kk in range(rt // CH):
            base = s * rt + kk * CH
            b = kk % PIPE
            pltpu.sync_copy(acc.at[pl.ds(base, CH)], rows_v.at[b])
            for cc in range(NC):
                @pl.when(c == cc)
                def _(cc=cc, b=b):
                    pltpu.sync_copy(rows_v.at[b],
                                    agg_outs[cc].at[pl.ds(base, CH)])

    return pl.kernel(body, out_type=out_type, mesh=mesh,
                     scratch_types=scratch,
                     compiler_params=pltpu.CompilerParams(
                         needs_layout_passes=False))


def _tc_layer(agg0, agg1, degp, h, wl, wr, b, br=1024):
    np_, d = h.shape

    def body(a0_ref, a1_ref, dg_ref, h_ref, wl_ref, wr_ref, b_ref,
             out_ref):
        a = a0_ref[...] + a1_ref[...]
        dg = jnp.sum(dg_ref[...], axis=0)[:, None]
        mean = a / jnp.maximum(dg, 1.0)
        acc = jnp.dot(mean, wl_ref[...], preferred_element_type=jnp.float32)
        acc += jnp.dot(h_ref[...], wr_ref[...],
                       preferred_element_type=jnp.float32)
        out_ref[...] = jnp.maximum(acc + b_ref[...], 0.0)

    return pl.pallas_call(
        body,
        grid=(np_ // br,),
        in_specs=[
            pl.BlockSpec((br, d), lambda i: (i, 0)),
            pl.BlockSpec((br, d), lambda i: (i, 0)),
            pl.BlockSpec((NW, br), lambda i: (0, i)),
            pl.BlockSpec((br, d), lambda i: (i, 0)),
            pl.BlockSpec((d, d), lambda i: (0, 0)),
            pl.BlockSpec((d, d), lambda i: (0, 0)),
            pl.BlockSpec((1, d), lambda i: (0, 0)),
        ],
        out_specs=pl.BlockSpec((br, d), lambda i: (i, 0)),
        out_shape=jax.ShapeDtypeStruct((np_, d), jnp.float32),
    )(agg0, agg1, degp, h, wl, wr, b)


def _tc_pool_mlp(h, batch_r, w1, b1, w2, b2, g, br=1024):
    np_, d = h.shape
    c = w2.shape[1]
    nblk = np_ // br

    def body(h_ref, batch_ref, w1_ref, b1_ref, w2_ref, b2_ref, out_ref,
             acc, cnt):
        i = pl.program_id(0)

        @pl.when(i == 0)
        def _():
            acc[...] = jnp.zeros_like(acc)
            cnt[...] = jnp.zeros_like(cnt)

        bt = batch_ref[0, 0, :]
        gid = lax.broadcasted_iota(jnp.int32, (g, br), 0)
        oh = (gid == bt[None, :]).astype(jnp.float32)
        acc[...] += jnp.dot(oh, h_ref[...],
                            preferred_element_type=jnp.float32)
        cnt[...] += jnp.sum(oh, axis=1, keepdims=True)

        @pl.when(i == nblk - 1)
        def _():
            pooled = acc[...] / jnp.maximum(cnt[...], 1.0)
            h2 = jnp.maximum(
                jnp.dot(pooled, w1_ref[...],
                        preferred_element_type=jnp.float32) + b1_ref[...],
                0.0)
            logits = jnp.dot(h2, w2_ref[...],
                             preferred_element_type=jnp.float32) + b2_ref[...]
            m = jnp.max(logits, axis=-1, keepdims=True)
            lse = jnp.log(jnp.sum(jnp.exp(logits - m), axis=-1,
                                  keepdims=True)) + m
            out_ref[...] = logits - lse

    return pl.pallas_call(
        body,
        grid=(nblk,),
        in_specs=[
            pl.BlockSpec((br, d), lambda i: (i, 0)),
            pl.BlockSpec((1, 1, br), lambda i: (i, 0, 0)),
            pl.BlockSpec((d, d), lambda i: (0, 0)),
            pl.BlockSpec((1, d), lambda i: (0, 0)),
            pl.BlockSpec((d, c), lambda i: (0, 0)),
            pl.BlockSpec((1, c), lambda i: (0, 0)),
        ],
        out_specs=pl.BlockSpec((g, c), lambda i: (0, 0)),
        out_shape=jax.ShapeDtypeStruct((g, c), jnp.float32),
        scratch_shapes=[
            pltpu.VMEM((g, d), jnp.float32),
            pltpu.VMEM((g, 1), jnp.float32),
        ],
    )(h, batch_r, w1, b1, w2, b2)


def kernel(x, edge_index, batch, Wl1, Wr1, b1, Wl2, Wr2, b2, Wl3, Wr3, b3,
           W_lin1, b_lin1, W_lin2, b_lin2):
    n, d = x.shape
    e = edge_index.shape[1]
    g = 64

    np_ = ((n + NS * CH - 1) // (NS * CH)) * (NS * CH)
    xp = jnp.zeros((np_, d), jnp.float32).at[:n].set(x)

    nch = (e + NW * CH - 1) // (NW * CH)
    nch = ((nch + PIPE - 1) // PIPE) * PIPE
    ep = NW * nch * CH
    src = jnp.concatenate(
        [edge_index[0], jnp.zeros((ep - e,), jnp.int32)]).reshape(
            ep // CH, CH)
    dst = jnp.concatenate(
        [edge_index[1], jnp.full((ep - e,), n, jnp.int32)]).reshape(
            ep // CH, CH)

    batch_r = jnp.concatenate(
        [batch, jnp.full((np_ - n,), g, jnp.int32)]).reshape(np_ // 1024, 1,
                                                             1024)

    b1r = b1.reshape(1, -1)
    b2r = b2.reshape(1, -1)
    b3r = b3.reshape(1, -1)
    bl1 = b_lin1.reshape(1, -1)
    bl2 = b_lin2.reshape(1, -1)

    zrows = jnp.zeros((CH, d), jnp.float32)
    zdeg = jnp.zeros((np_,), jnp.float32)

    sc_first = _make_sc_agg(np_, d, nch, want_deg=True)
    sc_rest = _make_sc_agg(np_, d, nch, want_deg=False)

    agg0, agg1, degp = sc_first(xp, src, dst, zrows, zdeg)
    h = _tc_layer(agg0, agg1, degp, xp, Wl1, Wr1, b1r)
    agg0, agg1 = sc_rest(h, src, dst, zrows)
    h = _tc_layer(agg0, agg1, degp, h, Wl2, Wr2, b2r)
    agg0, agg1 = sc_rest(h, src, dst, zrows)
    h = _tc_layer(agg0, agg1, degp, h, Wl3, Wr3, b3r)

    return _tc_pool_mlp(h, batch_r, W_lin1, bl1, W_lin2, bl2, g)


# speedup vs baseline: 1.0072x; 1.0072x over previous
import functools

import jax
import jax.numpy as jnp
from jax import lax
from jax.experimental import pallas as pl
from jax.experimental.pallas import tpu as pltpu
from jax.experimental.pallas import tpu_sc as plsc

NC = 2
NS = 16
LANES = 16
NW = NC * NS
CH = 128
PIPE = 2


def _make_sc_agg(np_, d, nch, want_deg):
    rt = np_ // NS
    assert rt % CH == 0

    mesh = plsc.VectorSubcoreMesh(core_axis_name="c", subcore_axis_name="s",
                                  num_cores=NC, num_subcores=NS)
    out_type = [jax.ShapeDtypeStruct((np_, d), jnp.float32)
                for _ in range(NC)]
    if want_deg:
        out_type.append(jax.ShapeDtypeStruct((NW, np_), jnp.float32))
    assert nch % PIPE == 0
    scratch = [
        pltpu.VMEM((PIPE, CH), jnp.int32),
        pltpu.VMEM((PIPE, CH), jnp.int32),
    ] + [pltpu.VMEM((CH, d), jnp.float32) for _ in range(PIPE)] + [
        pltpu.VMEM_SHARED((np_, d), jnp.float32),
    ] + [pltpu.SemaphoreType.DMA] * PIPE
    if want_deg:
        scratch.append(pltpu.VMEM((np_,), jnp.float32))

    def body(h_hbm, src_hbm, dst_hbm, zrows_hbm, *refs):
        refs = list(refs)
        if want_deg:
            zdeg_hbm = refs[0]
            agg_outs = refs[1:1 + NC]
            deg_out = refs[1 + NC]
            rest = refs[2 + NC:]
            degloc = rest[-1]
            rest = rest[:-1]
        else:
            agg_outs = refs[0:NC]
            rest = refs[NC:]
        src_v, dst_v = rest[0], rest[1]
        rows = rest[2:2 + PIPE]
        acc = rest[2 + PIPE]
        sems = rest[3 + PIPE:3 + 2 * PIPE]

        c = lax.axis_index("c")
        s = lax.axis_index("s")
        wid = s * NC + c

        pltpu.sync_copy(zrows_hbm, rows[0])
        for kk in range(rt // CH):
            base = s * rt + kk * CH
            pltpu.sync_copy(rows[0], acc.at[pl.ds(base, CH)])
        if want_deg:
            pltpu.sync_copy(zdeg_hbm, degloc)
        plsc.subcore_barrier()

        wrow = wid * nch
        ones_vec = jnp.ones((LANES,), jnp.float32)

        @pl.loop(0, nch // PIPE)
        def edge_group(i):
            r0 = wrow + i * PIPE
            pltpu.sync_copy(src_hbm.at[pl.ds(r0, PIPE)], src_v)
            pltpu.sync_copy(dst_hbm.at[pl.ds(r0, PIPE)], dst_v)
            copies = [
                pltpu.async_copy(h_hbm.at[src_v.at[b]], rows[b], sems[b])
                for b in range(PIPE)
            ]
            for b in range(PIPE):
                copies[b].wait()
                pltpu.sync_copy(rows[b], acc.at[dst_v.at[b]], add=True)
                if want_deg:
                    for kk in range(CH // LANES):
                        idx = dst_v[b, pl.ds(kk * LANES, LANES)]
                        plsc.addupdate_scatter(degloc, [idx], ones_vec)

        plsc.subcore_barrier()

        if want_deg:
            pltpu.sync_copy(degloc, deg_out.at[wid])

        for kk in range(rt // CH):
            base = s * rt + kk * CH
            b = kk % PIPE
            pltpu.sync_copy(acc.at[pl.ds(base, CH)], rows[b])
            for cc in range(NC):
                @pl.when(c == cc)
                def _(cc=cc, b=b):
                    pltpu.sync_copy(rows[b],
                                    agg_outs[cc].at[pl.ds(base, CH)])

    return pl.kernel(body, out_type=out_type, mesh=mesh,
                     scratch_types=scratch,
                     compiler_params=pltpu.CompilerParams(
                         needs_layout_passes=False))


def _tc_layer(agg0, agg1, degp, h, wl, wr, b, br=1024):
    np_, d = h.shape

    def body(a0_ref, a1_ref, dg_ref, h_ref, wl_ref, wr_ref, b_ref,
             out_ref):
        a = a0_ref[...] + a1_ref[...]
        dg = jnp.sum(dg_ref[...], axis=0)[:, None]
        mean = a / jnp.maximum(dg, 1.0)
        acc = jnp.dot(mean, wl_ref[...], preferred_element_type=jnp.float32)
        acc += jnp.dot(h_ref[...], wr_ref[...],
                       preferred_element_type=jnp.float32)
        out_ref[...] = jnp.maximum(acc + b_ref[...], 0.0)

    return pl.pallas_call(
        body,
        grid=(np_ // br,),
        in_specs=[
            pl.BlockSpec((br, d), lambda i: (i, 0)),
            pl.BlockSpec((br, d), lambda i: (i, 0)),
            pl.BlockSpec((NW, br), lambda i: (0, i)),
            pl.BlockSpec((br, d), lambda i: (i, 0)),
            pl.BlockSpec((d, d), lambda i: (0, 0)),
            pl.BlockSpec((d, d), lambda i: (0, 0)),
            pl.BlockSpec((1, d), lambda i: (0, 0)),
        ],
        out_specs=pl.BlockSpec((br, d), lambda i: (i, 0)),
        out_shape=jax.ShapeDtypeStruct((np_, d), jnp.float32),
    )(agg0, agg1, degp, h, wl, wr, b)


def _tc_pool_mlp(h, batch_r, w1, b1, w2, b2, g, br=1024):
    np_, d = h.shape
    c = w2.shape[1]
    nblk = np_ // br

    def body(h_ref, batch_ref, w1_ref, b1_ref, w2_ref, b2_ref, out_ref,
             acc, cnt):
        i = pl.program_id(0)

        @pl.when(i == 0)
        def _():
            acc[...] = jnp.zeros_like(acc)
            cnt[...] = jnp.zeros_like(cnt)

        bt = batch_ref[0, 0, :]
        gid = lax.broadcasted_iota(jnp.int32, (g, br), 0)
        oh = (gid == bt[None, :]).astype(jnp.float32)
        acc[...] += jnp.dot(oh, h_ref[...],
                            preferred_element_type=jnp.float32)
        cnt[...] += jnp.sum(oh, axis=1, keepdims=True)

        @pl.when(i == nblk - 1)
        def _():
            pooled = acc[...] / jnp.maximum(cnt[...], 1.0)
            h2 = jnp.maximum(
                jnp.dot(pooled, w1_ref[...],
                        preferred_element_type=jnp.float32) + b1_ref[...],
                0.0)
            logits = jnp.dot(h2, w2_ref[...],
                             preferred_element_type=jnp.float32) + b2_ref[...]
            m = jnp.max(logits, axis=-1, keepdims=True)
            lse = jnp.log(jnp.sum(jnp.exp(logits - m), axis=-1,
                                  keepdims=True)) + m
            out_ref[...] = logits - lse

    return pl.pallas_call(
        body,
        grid=(nblk,),
        in_specs=[
            pl.BlockSpec((br, d), lambda i: (i, 0)),
            pl.BlockSpec((1, 1, br), lambda i: (i, 0, 0)),
            pl.BlockSpec((d, d), lambda i: (0, 0)),
            pl.BlockSpec((1, d), lambda i: (0, 0)),
            pl.BlockSpec((d, c), lambda i: (0, 0)),
            pl.BlockSpec((1, c), lambda i: (0, 0)),
        ],
        out_specs=pl.BlockSpec((g, c), lambda i: (0, 0)),
        out_shape=jax.ShapeDtypeStruct((g, c), jnp.float32),
        scratch_shapes=[
            pltpu.VMEM((g, d), jnp.float32),
            pltpu.VMEM((g, 1), jnp.float32),
        ],
    )(h, batch_r, w1, b1, w2, b2)


def kernel(x, edge_index, batch, Wl1, Wr1, b1, Wl2, Wr2, b2, Wl3, Wr3, b3,
           W_lin1, b_lin1, W_lin2, b_lin2):
    n, d = x.shape
    e = edge_index.shape[1]
    g = 64

    np_ = ((n + NS * CH - 1) // (NS * CH)) * (NS * CH)
    xp = jnp.zeros((np_, d), jnp.float32).at[:n].set(x)

    nch = (e + NW * CH - 1) // (NW * CH)
    nch = ((nch + PIPE - 1) // PIPE) * PIPE
    ep = NW * nch * CH
    src = jnp.concatenate(
        [edge_index[0], jnp.zeros((ep - e,), jnp.int32)]).reshape(
            ep // CH, CH)
    dst = jnp.concatenate(
        [edge_index[1], jnp.full((ep - e,), n, jnp.int32)]).reshape(
            ep // CH, CH)

    batch_r = jnp.concatenate(
        [batch, jnp.full((np_ - n,), g, jnp.int32)]).reshape(np_ // 1024, 1,
                                                             1024)

    b1r = b1.reshape(1, -1)
    b2r = b2.reshape(1, -1)
    b3r = b3.reshape(1, -1)
    bl1 = b_lin1.reshape(1, -1)
    bl2 = b_lin2.reshape(1, -1)

    zrows = jnp.zeros((CH, d), jnp.float32)
    zdeg = jnp.zeros((np_,), jnp.float32)

    sc_first = _make_sc_agg(np_, d, nch, want_deg=True)
    sc_rest = _make_sc_agg(np_, d, nch, want_deg=False)

    agg0, agg1, degp = sc_first(xp, src, dst, zrows, zdeg)
    h = _tc_layer(agg0, agg1, degp, xp, Wl1, Wr1, b1r)
    agg0, agg1 = sc_rest(h, src, dst, zrows)
    h = _tc_layer(agg0, agg1, degp, h, Wl2, Wr2, b2r)
    agg0, agg1 = sc_rest(h, src, dst, zrows)
    h = _tc_layer(agg0, agg1, degp, h, Wl3, Wr3, b3r)

    return _tc_pool_mlp(h, batch_r, W_lin1, bl1, W_lin2, bl2, g)


# speedup vs baseline: 1.5078x; 1.4971x over previous
import functools

import jax
import jax.numpy as jnp
from jax import lax
from jax.experimental import pallas as pl
from jax.experimental.pallas import tpu as pltpu
from jax.experimental.pallas import tpu_sc as plsc

NC = 2
NS = 16
LANES = 16
NW = NC * NS
CH = 128
PIPE = 2


def _make_sc_agg(np_, d, nch, want_deg):
    rt = np_ // NS
    assert rt % CH == 0

    mesh = plsc.VectorSubcoreMesh(core_axis_name="c", subcore_axis_name="s",
                                  num_cores=NC, num_subcores=NS)
    out_type = [jax.ShapeDtypeStruct((np_, d), jnp.float32)
                for _ in range(NC)]
    if want_deg:
        out_type.append(jax.ShapeDtypeStruct((NW, np_), jnp.float32))
    scratch = [
        pltpu.VMEM((CH,), jnp.int32),
        pltpu.VMEM((1, CH), jnp.int32),
        pltpu.VMEM((CH, d), jnp.float32),
        pltpu.VMEM_SHARED((np_, d), jnp.float32),
        pltpu.SemaphoreType.DMA,
    ]
    if want_deg:
        scratch.append(pltpu.VMEM((np_,), jnp.float32))

    def body(h_hbm, src_hbm, dst_hbm, zrows_hbm, *refs):
        refs = list(refs)
        if want_deg:
            zdeg_hbm = refs[0]
            agg_outs = refs[1:1 + NC]
            deg_out = refs[1 + NC]
            src_v, dst_v, rows_v, acc, sem, degloc = refs[2 + NC:]
        else:
            agg_outs = refs[0:NC]
            src_v, dst_v, rows_v, acc, sem = refs[NC:]

        c = lax.axis_index("c")
        s = lax.axis_index("s")
        wid = s * NC + c

        pltpu.sync_copy(zrows_hbm, rows_v)
        for kk in range(rt // CH):
            base = s * rt + kk * CH
            pltpu.sync_copy(rows_v, acc.at[pl.ds(base, CH)])
        if want_deg:
            pltpu.sync_copy(zdeg_hbm, degloc)
        plsc.subcore_barrier()

        ebase = wid * nch * CH
        ones_vec = jnp.ones((LANES,), jnp.float32)

        @pl.loop(0, nch)
        def edge_chunk(j):
            eb = pl.multiple_of(ebase + j * CH, CH)
            pltpu.sync_copy(src_hbm.at[pl.ds(eb, CH)], src_v)
            pltpu.sync_copy(dst_hbm.at[pl.ds(eb, CH)], dst_v.at[0])
            pltpu.async_copy(h_hbm.at[src_v], rows_v, sem).wait()
            pltpu.sync_copy(rows_v, acc.at[dst_v.at[0]], add=True)
            if want_deg:
                for kk in range(CH // LANES):
                    idx = dst_v[0, pl.ds(kk * LANES, LANES)]
                    plsc.addupdate_scatter(degloc, [idx], ones_vec)

        plsc.subcore_barrier()

        if want_deg:
            pltpu.sync_copy(degloc, deg_out.at[wid])

        for kk in range(rt // CH):
            base = s * rt + kk * CH
            pltpu.sync_copy(acc.at[pl.ds(base, CH)], rows_v)
            for cc in range(NC):
                @pl.when(c == cc)
                def _(cc=cc):
                    pltpu.sync_copy(rows_v, agg_outs[cc].at[pl.ds(base, CH)])

    return pl.kernel(body, out_type=out_type, mesh=mesh,
                     scratch_types=scratch,
                     compiler_params=pltpu.CompilerParams(
                         needs_layout_passes=False))


def _tc_layer(agg0, agg1, degp, h, wl, wr, b, br=1024):
    np_, d = h.shape

    def body(a0_ref, a1_ref, dg_ref, h_ref, wl_ref, wr_ref, b_ref,
             out_ref):
        a = a0_ref[...] + a1_ref[...]
        dg = jnp.sum(dg_ref[...], axis=0)[:, None]
        mean = a / jnp.maximum(dg, 1.0)
        acc = jnp.dot(mean, wl_ref[...], preferred_element_type=jnp.float32)
        acc += jnp.dot(h_ref[...], wr_ref[...],
                       preferred_element_type=jnp.float32)
        out_ref[...] = jnp.maximum(acc + b_ref[...], 0.0)

    return pl.pallas_call(
        body,
        grid=(np_ // br,),
        in_specs=[
            pl.BlockSpec((br, d), lambda i: (i, 0)),
            pl.BlockSpec((br, d), lambda i: (i, 0)),
            pl.BlockSpec((NW, br), lambda i: (0, i)),
            pl.BlockSpec((br, d), lambda i: (i, 0)),
            pl.BlockSpec((d, d), lambda i: (0, 0)),
            pl.BlockSpec((d, d), lambda i: (0, 0)),
            pl.BlockSpec((1, d), lambda i: (0, 0)),
        ],
        out_specs=pl.BlockSpec((br, d), lambda i: (i, 0)),
        out_shape=jax.ShapeDtypeStruct((np_, d), jnp.float32),
    )(agg0, agg1, degp, h, wl, wr, b)


def _tc_pool_mlp(h, batch_r, w1, b1, w2, b2, g, br=1024):
    np_, d = h.shape
    c = w2.shape[1]
    nblk = np_ // br

    def body(h_ref, batch_ref, w1_ref, b1_ref, w2_ref, b2_ref, out_ref,
             acc, cnt):
        i = pl.program_id(0)

        @pl.when(i == 0)
        def _():
            acc[...] = jnp.zeros_like(acc)
            cnt[...] = jnp.zeros_like(cnt)

        bt = batch_ref[0, 0, :]
        gid = lax.broadcasted_iota(jnp.int32, (g, br), 0)
        oh = (gid == bt[None, :]).astype(jnp.float32)
        acc[...] += jnp.dot(oh, h_ref[...],
                            preferred_element_type=jnp.float32)
        cnt[...] += jnp.sum(oh, axis=1, keepdims=True)

        @pl.when(i == nblk - 1)
        def _():
            pooled = acc[...] / jnp.maximum(cnt[...], 1.0)
            h2 = jnp.maximum(
                jnp.dot(pooled, w1_ref[...],
                        preferred_element_type=jnp.float32) + b1_ref[...],
                0.0)
            logits = jnp.dot(h2, w2_ref[...],
                             preferred_element_type=jnp.float32) + b2_ref[...]
            m = jnp.max(logits, axis=-1, keepdims=True)
            lse = jnp.log(jnp.sum(jnp.exp(logits - m), axis=-1,
                                  keepdims=True)) + m
            out_ref[...] = logits - lse

    return pl.pallas_call(
        body,
        grid=(nblk,),
        in_specs=[
            pl.BlockSpec((br, d), lambda i: (i, 0)),
            pl.BlockSpec((1, 1, br), lambda i: (i, 0, 0)),
            pl.BlockSpec((d, d), lambda i: (0, 0)),
            pl.BlockSpec((1, d), lambda i: (0, 0)),
            pl.BlockSpec((d, c), lambda i: (0, 0)),
            pl.BlockSpec((1, c), lambda i: (0, 0)),
        ],
        out_specs=pl.BlockSpec((g, c), lambda i: (0, 0)),
        out_shape=jax.ShapeDtypeStruct((g, c), jnp.float32),
        scratch_shapes=[
            pltpu.VMEM((g, d), jnp.float32),
            pltpu.VMEM((g, 1), jnp.float32),
        ],
    )(h, batch_r, w1, b1, w2, b2)


def kernel(x, edge_index, batch, Wl1, Wr1, b1, Wl2, Wr2, b2, Wl3, Wr3, b3,
           W_lin1, b_lin1, W_lin2, b_lin2):
    n, d = x.shape
    e = edge_index.shape[1]
    g = 64

    np_ = ((n + NS * CH - 1) // (NS * CH)) * (NS * CH)
    xp = jnp.zeros((np_, d), jnp.float32).at[:n].set(x)

    nch = (e + NW * CH - 1) // (NW * CH)
    ep = NW * nch * CH
    src = jnp.concatenate(
        [edge_index[0], jnp.zeros((ep - e,), jnp.int32)])
    dst = jnp.concatenate(
        [edge_index[1], jnp.full((ep - e,), n, jnp.int32)])

    batch_r = jnp.concatenate(
        [batch, jnp.full((np_ - n,), g, jnp.int32)]).reshape(np_ // 1024, 1,
                                                             1024)

    b1r = b1.reshape(1, -1)
    b2r = b2.reshape(1, -1)
    b3r = b3.reshape(1, -1)
    bl1 = b_lin1.reshape(1, -1)
    bl2 = b_lin2.reshape(1, -1)

    zrows = jnp.zeros((CH, d), jnp.float32)
    zdeg = jnp.zeros((np_,), jnp.float32)

    sc_first = _make_sc_agg(np_, d, nch, want_deg=True)
    sc_rest = _make_sc_agg(np_, d, nch, want_deg=False)

    agg0, agg1, degp = sc_first(xp, src, dst, zrows, zdeg)
    h = _tc_layer(agg0, agg1, degp, xp, Wl1, Wr1, b1r)
    agg0, agg1 = sc_rest(h, src, dst, zrows)
    h = _tc_layer(agg0, agg1, degp, h, Wl2, Wr2, b2r)
    agg0, agg1 = sc_rest(h, src, dst, zrows)
    h = _tc_layer(agg0, agg1, degp, h, Wl3, Wr3, b3r)

    return _tc_pool_mlp(h, batch_r, W_lin1, bl1, W_lin2, bl2, g)
